# Initial kernel scaffold; baseline (speedup 1.0000x reference)
import jax
import jax.numpy as jnp
from jax.experimental import pallas as pl


def kernel(rpn_class, rpn_bbox, anchors):
    raise NotImplementedError("write your pallas kernel here")


# speedup vs baseline: 37.0189x; 37.0189x over previous
import functools

import jax
import jax.numpy as jnp
import numpy as np
from jax.experimental import pallas as pl
from jax.experimental.pallas import tpu as pltpu

_PROPOSALS = 1000
_THR = 0.7
_PRE = 6000
_PAD = 6144
_NB = 48
_BLK = 128
_OUTPAD = 1024
_STD = np.array([0.1, 0.1, 0.2, 0.2], dtype=np.float32)


def _decode_row(an, dl):
    ay1, ax1, ay2, ax2 = an[0:1], an[1:2], an[2:3], an[3:4]
    dy, dx, dh, dw = dl[0:1], dl[1:2], dl[2:3], dl[3:4]
    h = ay2 - ay1
    w = ax2 - ax1
    cy = ay1 + 0.5 * h
    cx = ax1 + 0.5 * w
    cy = cy + dy * h
    cx = cx + dx * w
    h = h * jnp.exp(dh)
    w = w * jnp.exp(dw)
    y1 = jnp.clip(cy - 0.5 * h, 0.0, 1.0)
    x1 = jnp.clip(cx - 0.5 * w, 0.0, 1.0)
    y2 = jnp.clip(cy - 0.5 * h + h, 0.0, 1.0)
    x2 = jnp.clip(cx - 0.5 * w + w, 0.0, 1.0)
    return y1, x1, y2, x2


def _nms_kernel(an_c_ref, dl_c_ref, an_r_ref, dl_r_ref, out_ref,
                box_r, box_c, area_r, area_c, keep_r, keep2, rank_s):
    f32 = jnp.float32

    y1, x1, y2, x2 = _decode_row(an_r_ref[...], dl_r_ref[...])
    box_r[0:1, :] = y1
    box_r[1:2, :] = x1
    box_r[2:3, :] = y2
    box_r[3:4, :] = x2
    area_r[...] = (y2 - y1) * (x2 - x1)

    an_c = an_c_ref[...]
    dl_c = dl_c_ref[...]
    ay1, ax1, ay2, ax2 = an_c[:, 0:1], an_c[:, 1:2], an_c[:, 2:3], an_c[:, 3:4]
    dy, dx, dh, dw = dl_c[:, 0:1], dl_c[:, 1:2], dl_c[:, 2:3], dl_c[:, 3:4]
    h = ay2 - ay1
    w = ax2 - ax1
    cy = ay1 + 0.5 * h + dy * h
    cx = ax1 + 0.5 * w + dx * w
    h = h * jnp.exp(dh)
    w = w * jnp.exp(dw)
    cy1 = jnp.clip(cy - 0.5 * h, 0.0, 1.0)
    cx1 = jnp.clip(cx - 0.5 * w, 0.0, 1.0)
    cy2 = jnp.clip(cy - 0.5 * h + h, 0.0, 1.0)
    cx2 = jnp.clip(cx - 0.5 * w + w, 0.0, 1.0)
    box_c[:, 0:1] = cy1
    box_c[:, 1:2] = cx1
    box_c[:, 2:3] = cy2
    box_c[:, 3:4] = cx2
    area_c[...] = (cy2 - cy1) * (cx2 - cx1)

    lane = jax.lax.broadcasted_iota(jnp.int32, (1, _PAD), 1)
    keep_r[...] = jnp.where(lane < _PRE, 1.0, 0.0).astype(f32)

    rj = jax.lax.broadcasted_iota(jnp.int32, (_BLK, _BLK), 0)
    ci = jax.lax.broadcasted_iota(jnp.int32, (_BLK, _BLK), 1)
    strict = rj < ci

    def iou_tile(b, c):
        yb1 = box_c[pl.ds(b * _BLK, _BLK), 0:1]
        xb1 = box_c[pl.ds(b * _BLK, _BLK), 1:2]
        yb2 = box_c[pl.ds(b * _BLK, _BLK), 2:3]
        xb2 = box_c[pl.ds(b * _BLK, _BLK), 3:4]
        ab = area_c[pl.ds(b * _BLK, _BLK), 0:1]
        yc1 = box_r[0:1, pl.ds(c * _BLK, _BLK)]
        xc1 = box_r[1:2, pl.ds(c * _BLK, _BLK)]
        yc2 = box_r[2:3, pl.ds(c * _BLK, _BLK)]
        xc2 = box_r[3:4, pl.ds(c * _BLK, _BLK)]
        ac = area_r[0:1, pl.ds(c * _BLK, _BLK)]
        iy1 = jnp.maximum(yb1, yc1)
        ix1 = jnp.maximum(xb1, xc1)
        iy2 = jnp.minimum(yb2, yc2)
        ix2 = jnp.minimum(xb2, xc2)
        inter = jnp.maximum(iy2 - iy1, 0.0) * jnp.maximum(ix2 - ix1, 0.0)
        union = ab + ac - inter
        return inter / jnp.maximum(union, 1e-8)

    def block_body(b, _):
        pre = keep_r[0:1, pl.ds(b * _BLK, _BLK)]
        iou_d = iou_tile(b, b)
        m_d = jnp.where((iou_d > _THR) & strict, 1.0, 0.0).astype(f32)

        def cond(carry):
            return carry[1]

        def body(carry):
            k = carry[0]
            dom = jnp.dot(k, m_d, preferred_element_type=f32)
            kn = pre * jnp.where(dom > 0.0, 0.0, 1.0)
            return (kn, jnp.any(kn != k))

        kf, _unused = jax.lax.while_loop(cond, body, (pre, True))
        keep_r[0:1, pl.ds(b * _BLK, _BLK)] = kf
        keep2[pl.ds(b, 1), :] = kf

        def cross(c, __):
            iou_x = iou_tile(b, c)
            m_x = jnp.where(iou_x > _THR, 1.0, 0.0).astype(f32)
            sup = jnp.dot(kf, m_x, preferred_element_type=f32)
            cur = keep_r[0:1, pl.ds(c * _BLK, _BLK)]
            new = cur * jnp.where(sup > 0.0, 0.0, 1.0)
            keep_r[0:1, pl.ds(c * _BLK, _BLK)] = new
            return __

        jax.lax.fori_loop(b + 1, _NB, cross, None)
        return _

    jax.lax.fori_loop(0, _NB, block_body, None)

    k2 = keep2[...]
    upper_inc = jnp.where(rj <= ci, 1.0, 0.0).astype(f32)
    incl = jnp.dot(k2, upper_inc, preferred_element_type=f32)
    totals = jnp.sum(k2, axis=1, keepdims=True)
    rb = jax.lax.broadcasted_iota(jnp.int32, (_NB, _NB), 0)
    cb = jax.lax.broadcasted_iota(jnp.int32, (_NB, _NB), 1)
    lower_strict = jnp.where(cb < rb, 1.0, 0.0).astype(f32)
    offs = jnp.dot(lower_strict, totals, preferred_element_type=f32)
    rank_s[...] = incl - k2 + offs

    out_ref[...] = jnp.zeros((_OUTPAD, 4), f32)
    slot = jax.lax.broadcasted_iota(jnp.int32, (_OUTPAD, 1), 0).astype(f32)

    def emit_body(c, _):
        rrow = rank_s[pl.ds(c, 1), :]
        krow = keep2[pl.ds(c, 1), :]
        sel = jnp.where((slot == rrow) & (krow > 0.0), 1.0, 0.0).astype(f32)
        tile = box_c[pl.ds(c * _BLK, _BLK), :]
        out_ref[...] += jnp.dot(sel, tile, preferred_element_type=f32)
        return _

    jax.lax.fori_loop(0, _NB, emit_body, None)


def _proposals_one(rpn_class, rpn_bbox, anchors):
    scores = rpn_class[:, 1]
    deltas = rpn_bbox * jnp.asarray(_STD).reshape(1, 4)
    _unused, ix = jax.lax.top_k(scores, _PRE)
    an = jnp.take(anchors, ix, axis=0)
    dl = jnp.take(deltas, ix, axis=0)
    an_c = jnp.pad(an, ((0, _PAD - _PRE), (0, 0)))
    dl_c = jnp.pad(dl, ((0, _PAD - _PRE), (0, 0)))
    an_r = an_c.T
    dl_r = dl_c.T
    out = pl.pallas_call(
        _nms_kernel,
        out_shape=jax.ShapeDtypeStruct((_OUTPAD, 4), jnp.float32),
        scratch_shapes=[
            pltpu.VMEM((4, _PAD), jnp.float32),
            pltpu.VMEM((_PAD, 4), jnp.float32),
            pltpu.VMEM((1, _PAD), jnp.float32),
            pltpu.VMEM((_PAD, 1), jnp.float32),
            pltpu.VMEM((1, _PAD), jnp.float32),
            pltpu.VMEM((_NB, _BLK), jnp.float32),
            pltpu.VMEM((_NB, _BLK), jnp.float32),
        ],
    )(an_c, dl_c, an_r, dl_r)
    return out[:_PROPOSALS]


def kernel(rpn_class, rpn_bbox, anchors):
    outs = [
        _proposals_one(rpn_class[b], rpn_bbox[b], anchors[b])
        for b in range(rpn_class.shape[0])
    ]
    return jnp.stack(outs, axis=0)
